# baseline (speedup 1.0000x reference)
import functools

import jax
import jax.numpy as jnp
from jax import lax
from jax.experimental import pallas as pl
from jax.experimental.pallas import tpu as pltpu
from jax.experimental.pallas import tpu_sc as plsc

_TIME_SCALE = 30.0
_B, _H, _W = 4096, 224, 224
_HW = _H * _W

_INFO = plsc.get_sparse_core_info()
_NC, _NS, _L = _INFO.num_cores, _INFO.num_subcores, _INFO.num_lanes
_NW = _NC * _NS
_BPW = _B // _NW
_CHUNKS = _BPW // _L

_mesh = plsc.VectorSubcoreMesh(core_axis_name="c", subcore_axis_name="s")


@functools.partial(
    pl.kernel,
    mesh=_mesh,
    out_type=jax.ShapeDtypeStruct((_B,), jnp.float32),
    scratch_types=[
        pltpu.VMEM((_BPW,), jnp.int32),
        pltpu.VMEM((_BPW,), jnp.int32),
        pltpu.VMEM((_BPW,), jnp.int32),
        pltpu.VMEM((_BPW,), jnp.float32),
        pltpu.SemaphoreType.DMA,
    ],
)
def _sc_gather(img_hbm, x_hbm, y_hbm, out_hbm, x_v, y_v, idx_v, vals_v, sem):
    wid = lax.axis_index("s") * _NC + lax.axis_index("c")
    base = wid * _BPW

    pltpu.sync_copy(x_hbm.at[pl.ds(base, _BPW)], x_v)
    pltpu.sync_copy(y_hbm.at[pl.ds(base, _BPW)], y_v)

    lane = lax.iota(jnp.int32, _L)
    for j in range(_CHUNKS):
        sl = pl.ds(_L * j, _L)
        b = base + _L * j + lane
        idx_v[sl] = b * _HW + y_v[sl] * _W + x_v[sl]

    pltpu.async_copy(img_hbm.at[idx_v], vals_v, sem).wait()

    for j in range(_CHUNKS):
        sl = pl.ds(_L * j, _L)
        vals_v[sl] = vals_v[sl] * _TIME_SCALE

    pltpu.sync_copy(vals_v, out_hbm.at[pl.ds(base, _BPW)])


def kernel(v_image, actor_pixel_selection):
    img = v_image.reshape(-1)
    x = actor_pixel_selection[:, 0]
    y = actor_pixel_selection[:, 1]
    out = _sc_gather(img, x, y)
    return out.reshape(_B, 1, 1)


# speedup vs baseline: 2.0168x; 2.0168x over previous
import functools

import jax
import jax.numpy as jnp
from jax import lax
from jax.experimental import pallas as pl
from jax.experimental.pallas import tpu as pltpu
from jax.experimental.pallas import tpu_sc as plsc

_TIME_SCALE = 30.0
_B, _H, _W = 4096, 224, 224

_INFO = plsc.get_sparse_core_info()
_NC, _NS, _L = _INFO.num_cores, _INFO.num_subcores, _INFO.num_lanes
_NW = _NC * _NS
_BPW = _B // _NW
_CHUNKS = _BPW // _L
_RND = 64

_mesh = plsc.VectorSubcoreMesh(core_axis_name="c", subcore_axis_name="s")


@functools.partial(
    pl.kernel,
    mesh=_mesh,
    compiler_params=pltpu.CompilerParams(
        needs_layout_passes=False, disable_bounds_checks=True),
    out_type=jax.ShapeDtypeStruct((_B,), jnp.float32),
    scratch_types=[
        pltpu.VMEM((_BPW,), jnp.int32),
        pltpu.VMEM((_BPW,), jnp.int32),
        pltpu.VMEM((_RND, 8, 128), jnp.float32),
        pltpu.VMEM((_BPW,), jnp.float32),
        pltpu.SemaphoreType.DMA,
    ],
)
def _sc_gather(img_hbm, x_hbm, y_hbm, out_hbm, x_v, y_v, blk_v, vals_v, sem):
    wid = lax.axis_index("s") * _NC + lax.axis_index("c")
    base = wid * _BPW

    pltpu.sync_copy(x_hbm.at[pl.ds(base, _BPW)], x_v)
    pltpu.sync_copy(y_hbm.at[pl.ds(base, _BPW)], y_v)

    lane = lax.iota(jnp.int32, _L)
    for rnd in range(_BPW // _RND):
        copies = []
        for g in range(_RND // _L):
            i0 = rnd * _RND + g * _L
            xg = x_v[pl.ds(i0, _L)]
            yg = y_v[pl.ds(i0, _L)]
            for r in range(_L):
                ys = pl.multiple_of((yg[r] // 8) * 8, 8)
                xo = pl.multiple_of((xg[r] // 128) * 128, 128)
                copies.append(pltpu.async_copy(
                    img_hbm.at[base + i0 + r, pl.ds(ys, 8), pl.ds(xo, 128)],
                    blk_v.at[g * _L + r], sem))
        for c in copies:
            c.wait()
        for g in range(_RND // _L):
            i0 = rnd * _RND + g * _L
            sl = pl.ds(i0, _L)
            xg = x_v[sl]
            yg = y_v[sl]
            picked = plsc.load_gather(
                blk_v, [g * _L + lane, yg % 8, xg % 128])
            vals_v[sl] = picked * _TIME_SCALE

    pltpu.sync_copy(vals_v, out_hbm.at[pl.ds(base, _BPW)])


def kernel(v_image, actor_pixel_selection):
    x = actor_pixel_selection[:, 0]
    y = actor_pixel_selection[:, 1]
    out = _sc_gather(v_image, x, y)
    return out.reshape(_B, 1, 1)


# speedup vs baseline: 2.7101x; 1.3438x over previous
import functools

import jax
import jax.numpy as jnp
from jax import lax
from jax.experimental import pallas as pl
from jax.experimental.pallas import tpu as pltpu
from jax.experimental.pallas import tpu_sc as plsc

_TIME_SCALE = 30.0
_B, _H, _W = 4096, 224, 224

_INFO = plsc.get_sparse_core_info()
_NC, _NS, _L = _INFO.num_cores, _INFO.num_subcores, _INFO.num_lanes
_NW = _NC * _NS
_BPW = _B // _NW
_CHUNKS = _BPW // _L

_mesh = plsc.VectorSubcoreMesh(core_axis_name="c", subcore_axis_name="s")


@functools.partial(
    pl.kernel,
    mesh=_mesh,
    compiler_params=pltpu.CompilerParams(
        needs_layout_passes=False, disable_bounds_checks=True),
    out_type=jax.ShapeDtypeStruct((_B,), jnp.float32),
    scratch_types=[
        pltpu.VMEM((_BPW,), jnp.int32),
        pltpu.VMEM((_BPW,), jnp.int32),
        pltpu.VMEM((_BPW,), jnp.int32),
        pltpu.VMEM((_BPW, 256), jnp.float32),
        pltpu.VMEM((_BPW,), jnp.float32),
        pltpu.SemaphoreType.DMA,
    ],
)
def _sc_gather(img_hbm, x_hbm, y_hbm, out_hbm, x_v, y_v, row_v, rows_v,
               vals_v, sem):
    wid = lax.axis_index("s") * _NC + lax.axis_index("c")
    base = wid * _BPW

    pltpu.sync_copy(x_hbm.at[pl.ds(base, _BPW)], x_v)
    pltpu.sync_copy(y_hbm.at[pl.ds(base, _BPW)], y_v)

    lane = lax.iota(jnp.int32, _L)
    for j in range(_CHUNKS):
        sl = pl.ds(_L * j, _L)
        b = base + _L * j + lane
        row_v[sl] = b * _H + y_v[sl]

    xo = pl.multiple_of(jnp.sum(x_v[pl.ds(0, _L)]) * 0 + 128, 128)

    c1 = pltpu.async_copy(
        img_hbm.at[row_v, pl.ds(0, 128)], rows_v.at[:, pl.ds(0, 128)], sem)
    c2 = pltpu.async_copy(
        img_hbm.at[row_v, pl.ds(xo, 128)], rows_v.at[:, pl.ds(128, 128)], sem)
    c1.wait()
    c2.wait()

    for g in range(_CHUNKS):
        sl = pl.ds(_L * g, _L)
        picked = plsc.load_gather(rows_v, [g * _L + lane, x_v[sl]])
        vals_v[sl] = picked * _TIME_SCALE

    pltpu.sync_copy(vals_v, out_hbm.at[pl.ds(base, _BPW)])


def kernel(v_image, actor_pixel_selection):
    img = v_image.reshape(_B * _H, _W)
    x = actor_pixel_selection[:, 0]
    y = actor_pixel_selection[:, 1]
    out = _sc_gather(img, x, y)
    return out.reshape(_B, 1, 1)
